# baseline (speedup 1.0000x reference)
import functools

import jax
import jax.numpy as jnp
from jax import lax
from jax.experimental import pallas as pl
from jax.experimental.pallas import tpu as pltpu
from jax.experimental.pallas import tpu_sc as plsc


def _add_kernel(x_ref, pe_ref, o_ref):
    o_ref[...] = x_ref[...] + pe_ref[...]


def _kernel_tc(x, pe):
    B, T, D = x.shape
    BS = 2048
    grid = (T // BS, B)
    return pl.pallas_call(
        _add_kernel,
        grid=grid,
        in_specs=[
            pl.BlockSpec((1, BS, D), lambda i, j: (j, i, 0)),
            pl.BlockSpec((BS, D), lambda i, j: (i, 0)),
        ],
        out_specs=pl.BlockSpec((1, BS, D), lambda i, j: (j, i, 0)),
        out_shape=jax.ShapeDtypeStruct((B, T, D), x.dtype),
        compiler_params=pltpu.CompilerParams(
            dimension_semantics=("parallel", "parallel"),
        ),
    )(x, pe[:T])


_NW = 32
_R = 32


def _kernel_sc(x, pe):
    B, T, D = x.shape
    N = B * T
    xf = x.reshape(N * D)
    pef = pe[:T].reshape(T * D)
    mesh = plsc.VectorSubcoreMesh(core_axis_name="c", subcore_axis_name="s")
    seq_per_w = T // _NW
    CH = _R * D

    @functools.partial(
        pl.kernel,
        mesh=mesh,
        out_type=jax.ShapeDtypeStruct((N * D,), jnp.float32),
        scratch_types=[
            pltpu.VMEM((CH,), jnp.float32),
            pltpu.VMEM((CH,), jnp.float32),
        ],
    )
    def k(x_hbm, pe_hbm, out_hbm, buf_x, buf_pe):
        wid = lax.axis_index("s") * 2 + lax.axis_index("c")

        def add_pe(v, _):
            o = v * 16
            plsc.addupdate(buf_x.at[pl.ds(o, 16)], buf_pe[pl.ds(o, 16)])
            return _

        for i in range(seq_per_w // _R):
            t0 = (wid * seq_per_w + i * _R) * D
            pltpu.sync_copy(pe_hbm.at[pl.ds(t0, CH)], buf_pe)
            for b in range(B):
                e0 = b * T * D + t0
                pltpu.sync_copy(x_hbm.at[pl.ds(e0, CH)], buf_x)
                lax.fori_loop(0, CH // 16, add_pe, None)
                pltpu.sync_copy(buf_x, out_hbm.at[pl.ds(e0, CH)])

    return k(xf, pef).reshape(B, T, D)


def kernel(x, pe):
    return _kernel_sc(x, pe)


# speedup vs baseline: 7.1510x; 7.1510x over previous
import functools

import jax
import jax.numpy as jnp
from jax import lax
from jax.experimental import pallas as pl
from jax.experimental.pallas import tpu as pltpu
from jax.experimental.pallas import tpu_sc as plsc


def _add_kernel2(x1_ref, x2_ref, pe1_ref, pe2_ref, o_ref):
    h = x1_ref.shape[-1]
    o_ref[:, :, :h] = x1_ref[...] + pe1_ref[...]
    o_ref[:, :, h:] = x2_ref[...] + pe2_ref[...]


def _kernel_tc(x, pe):
    B, T, D = x.shape
    BS = 2048
    H = D // 2
    grid = (T // BS, B)
    return pl.pallas_call(
        _add_kernel2,
        grid=grid,
        in_specs=[
            pl.BlockSpec((1, BS, H), lambda i, j: (j, i, 0)),
            pl.BlockSpec((1, BS, H), lambda i, j: (j, i, 1)),
            pl.BlockSpec((BS, H), lambda i, j: (i, 0)),
            pl.BlockSpec((BS, H), lambda i, j: (i, 1)),
        ],
        out_specs=pl.BlockSpec((1, BS, D), lambda i, j: (j, i, 0)),
        out_shape=jax.ShapeDtypeStruct((B, T, D), x.dtype),
        compiler_params=pltpu.CompilerParams(
            dimension_semantics=("parallel", "parallel"),
        ),
    )(x, x, pe[:T], pe[:T])


_NW = 32
_R = 32


def _kernel_sc(x, pe):
    B, T, D = x.shape
    N = B * T
    xf = x.reshape(N * D)
    pef = pe[:T].reshape(T * D)
    mesh = plsc.VectorSubcoreMesh(core_axis_name="c", subcore_axis_name="s")
    seq_per_w = T // _NW
    CH = _R * D

    @functools.partial(
        pl.kernel,
        mesh=mesh,
        out_type=jax.ShapeDtypeStruct((N * D,), jnp.float32),
        scratch_types=[
            pltpu.VMEM((CH,), jnp.float32),
            pltpu.VMEM((CH,), jnp.float32),
        ],
    )
    def k(x_hbm, pe_hbm, out_hbm, buf_x, buf_pe):
        wid = lax.axis_index("s") * 2 + lax.axis_index("c")

        def add_pe(v, _):
            o = v * 16
            plsc.addupdate(buf_x.at[pl.ds(o, 16)], buf_pe[pl.ds(o, 16)])
            return _

        for i in range(seq_per_w // _R):
            t0 = (wid * seq_per_w + i * _R) * D
            pltpu.sync_copy(pe_hbm.at[pl.ds(t0, CH)], buf_pe)
            for b in range(B):
                e0 = b * T * D + t0
                pltpu.sync_copy(x_hbm.at[pl.ds(e0, CH)], buf_x)
                lax.fori_loop(0, CH // 16, add_pe, None)
                pltpu.sync_copy(buf_x, out_hbm.at[pl.ds(e0, CH)])

    return k(xf, pef).reshape(B, T, D)


def kernel(x, pe):
    return _kernel_tc(x, pe)


# speedup vs baseline: 7.1934x; 1.0059x over previous
import functools

import jax
import jax.numpy as jnp
from jax import lax
from jax.experimental import pallas as pl
from jax.experimental.pallas import tpu as pltpu
from jax.experimental.pallas import tpu_sc as plsc


def _add_kernel(x_ref, pe_ref, o_ref):
    o_ref[...] = x_ref[...] + pe_ref[...]


def _kernel_tc(x, pe):
    B, T, D = x.shape
    BS = 2048
    grid = (T // BS, B)
    return pl.pallas_call(
        _add_kernel,
        grid=grid,
        in_specs=[
            pl.BlockSpec((1, BS, D), lambda i, j: (j, i, 0)),
            pl.BlockSpec((BS, D), lambda i, j: (i, 0)),
        ],
        out_specs=pl.BlockSpec((1, BS, D), lambda i, j: (j, i, 0)),
        out_shape=jax.ShapeDtypeStruct((B, T, D), x.dtype),
        compiler_params=pltpu.CompilerParams(
            dimension_semantics=("parallel", "parallel"),
        ),
    )(x, pe[:T])


_NW = 32
_R = 32


def _kernel_sc(x, pe):
    B, T, D = x.shape
    N = B * T
    xf = x.reshape(N * D)
    pef = pe[:T].reshape(T * D)
    mesh = plsc.VectorSubcoreMesh(core_axis_name="c", subcore_axis_name="s")
    seq_per_w = T // _NW
    CH = _R * D

    @functools.partial(
        pl.kernel,
        mesh=mesh,
        out_type=jax.ShapeDtypeStruct((N * D,), jnp.float32),
        scratch_types=[
            pltpu.VMEM((CH,), jnp.float32),
            pltpu.VMEM((CH,), jnp.float32),
        ],
    )
    def k(x_hbm, pe_hbm, out_hbm, buf_x, buf_pe):
        wid = lax.axis_index("s") * 2 + lax.axis_index("c")

        def add_pe(v, _):
            o = v * 16
            plsc.addupdate(buf_x.at[pl.ds(o, 16)], buf_pe[pl.ds(o, 16)])
            return _

        for i in range(seq_per_w // _R):
            t0 = (wid * seq_per_w + i * _R) * D
            pltpu.sync_copy(pe_hbm.at[pl.ds(t0, CH)], buf_pe)
            for b in range(B):
                e0 = b * T * D + t0
                pltpu.sync_copy(x_hbm.at[pl.ds(e0, CH)], buf_x)
                lax.fori_loop(0, CH // 16, add_pe, None)
                pltpu.sync_copy(buf_x, out_hbm.at[pl.ds(e0, CH)])

    return k(xf, pef).reshape(B, T, D)


def kernel(x, pe):
    return _kernel_tc(x, pe)
